# baseline (speedup 1.0000x reference)
import functools

import jax
import jax.numpy as jnp
import numpy as np
from jax import lax
from jax.experimental import pallas as pl
from jax.experimental.pallas import tpu as pltpu
from jax.experimental.pallas import tpu_sc as plsc

N = 10000
E = 320000
D = 128
H = 256

NC = 2
NS = 16
NW = NC * NS
CHUNK = 128
CPT = 80
NCHUNKS = NW * CPT
EPAD = NCHUNKS * CHUNK
NBUF = 1
NACC = 10240
RPT = NACC // NS

BN = 1000
NB = N // BN
_BN_SCALE = np.float32(1.0 / np.sqrt(1.0 + 1e-5))

_mesh = plsc.VectorSubcoreMesh(
    core_axis_name="c", subcore_axis_name="s", num_cores=NC, num_subcores=NS)


def _zero_rows_v(rows_v, nrow, ncol):
    def body(i, _):
        for j in range(ncol // 16):
            rows_v[i, pl.ds(j * 16, 16)] = jnp.zeros((16,), jnp.float32)
        return _
    lax.fori_loop(0, nrow, body, None, unroll=False)


def _sc_deg_body(dst_hbm, out_hbm, acc, idx_v, ones_v, zero_v):
    c = lax.axis_index("c")
    s = lax.axis_index("s")
    w = c * NS + s
    one0 = jnp.where(lax.iota(jnp.int32, 16) == 0,
                     jnp.float32(1.0), jnp.float32(0.0))

    def initrow(i, _):
        ones_v[i, pl.ds(0, 16)] = one0
        zero_v[i, pl.ds(0, 16)] = jnp.zeros((16,), jnp.float32)
        return _
    lax.fori_loop(0, CHUNK, initrow, None, unroll=False)

    zb = s * RPT
    for t in range(RPT // CHUNK):
        pltpu.sync_copy(zero_v, acc.at[pl.ds(zb + t * CHUNK, CHUNK)])
    plsc.subcore_barrier()

    def chunk(k, _):
        cid = w * CPT + k
        pltpu.sync_copy(dst_hbm.at[cid], idx_v)
        pltpu.sync_copy(ones_v, acc.at[idx_v.at[0]], add=True)
        return _
    lax.fori_loop(0, CPT, chunk, None, unroll=False)
    plsc.subcore_barrier()
    pltpu.sync_copy(acc.at[pl.ds(s * RPT, RPT)],
                    out_hbm.at[c, pl.ds(s * RPT, RPT)])


@functools.partial(
    pl.kernel,
    out_type=jax.ShapeDtypeStruct((NC, NACC, 16), jnp.float32),
    mesh=_mesh,
    scratch_types=[
        pltpu.VMEM_SHARED((NACC, 16), jnp.float32),
        pltpu.VMEM((1, CHUNK), jnp.int32),
        pltpu.VMEM((CHUNK, 16), jnp.float32),
        pltpu.VMEM((CHUNK, 16), jnp.float32),
    ],
)
def _sc_deg(dst_hbm, out_hbm, acc, idx_v, ones_v, zero_v):
    _sc_deg_body(dst_hbm, out_hbm, acc, idx_v, ones_v, zero_v)


CPT0 = 80
CPT1 = 80


def _sc_agg_body(hs_hbm, src_hbm, dst_hbm, out_hbm, acc,
                 src_v, dst_v, rows_v, sem):
    c = lax.axis_index("c")
    s = lax.axis_index("s")
    w = c * NS + s
    _zero_rows_v(rows_v, CHUNK, D)
    zb = s * RPT
    for t in range(RPT // CHUNK):
        pltpu.sync_copy(rows_v, acc.at[pl.ds(zb + t * CHUNK, CHUNK)])
    plsc.subcore_barrier()

    def chunk(k, _):
        cid = w * CPT + k
        pltpu.sync_copy(src_hbm.at[cid], src_v)
        pltpu.sync_copy(dst_hbm.at[cid], dst_v)
        pltpu.async_copy(hs_hbm.at[src_v.at[0]], rows_v, sem).wait()
        pltpu.sync_copy(rows_v, acc.at[dst_v.at[0]], add=True)
        return _
    lax.fori_loop(0, CPT, chunk, None, unroll=False)
    plsc.subcore_barrier()
    pltpu.sync_copy(acc.at[pl.ds(s * RPT, RPT)],
                    out_hbm.at[c, pl.ds(s * RPT, RPT)])


@functools.partial(
    pl.kernel,
    out_type=jax.ShapeDtypeStruct((NC, NACC, D), jnp.float32),
    mesh=_mesh,
    scratch_types=[
        pltpu.VMEM_SHARED((NACC, D), jnp.float32),
        pltpu.VMEM((1, CHUNK), jnp.int32),
        pltpu.VMEM((1, CHUNK), jnp.int32),
        pltpu.VMEM((CHUNK, D), jnp.float32),
        pltpu.SemaphoreType.DMA,
    ],
)
def _sc_agg(hs_hbm, src_hbm, dst_hbm, out_hbm, acc, src_v, dst_v, rows_v, sem):
    _sc_agg_body(hs_hbm, src_hbm, dst_hbm, out_hbm, acc,
                 src_v, dst_v, rows_v, sem)


def _dinv(p0, p1):
    deg = 1.0 + p0[0][:, 0:1] + p1[0][:, 0:1]
    return lax.rsqrt(deg)


def _tc_prep_body(x_ref, p0_ref, p1_ref, hs1_ref):
    hs1_ref[...] = x_ref[...] * _dinv(p0_ref[...], p1_ref[...])


def _tc_l1_body(a0, a1, hs1, p0, p1, w1, b1, g1, c1, oa, ob):
    dinv = _dinv(p0[...], p1[...])
    comb = (a0[0] + a1[0] + hs1[...]) * dinv
    z = jnp.dot(comb, w1[...], preferred_element_type=jnp.float32) + b1[...]
    h = jax.nn.relu(z * (g1[...] * _BN_SCALE) + c1[...])
    hs2 = h * dinv
    oa[...] = hs2[:, :D]
    ob[...] = hs2[:, D:]


def _tc_l2_body(aa0, aa1, ab0, ab1, hsa, hsb, p0, p1, w2, b2, g2, c2, w3, o):
    dinv = _dinv(p0[...], p1[...])
    comb = jnp.concatenate(
        [aa0[0] + aa1[0] + hsa[...], ab0[0] + ab1[0] + hsb[...]],
        axis=1) * dinv
    z = jnp.dot(comb, w2[...], preferred_element_type=jnp.float32) + b2[...]
    h = jax.nn.relu(z * (g2[...] * _BN_SCALE) + c2[...])
    m3 = jnp.dot(h, w3[...], preferred_element_type=jnp.float32)
    o[...] = m3 * dinv


def _tc_fin_body(a0, a1, hs3, p0, p1, b3, o):
    dinv = _dinv(p0[...], p1[...])
    o[...] = (a0[0] + a1[0] + hs3[...]) * dinv + b3[...]


def _part0(i):
    return (0, i, 0)


def _part1(i):
    return (1, i, 0)


def _rows(i):
    return (i, 0)


def _fixed(i):
    return (0, 0)


def _fixed1(i):
    return (0,)


_spec_row = pl.BlockSpec((BN, D), _rows)
_spec_a0 = pl.BlockSpec((1, BN, D), _part0)
_spec_a1 = pl.BlockSpec((1, BN, D), _part1)
_spec_p0 = pl.BlockSpec((1, BN, 16), _part0)
_spec_p1 = pl.BlockSpec((1, BN, 16), _part1)


def _tc_prep(x, degp):
    return pl.pallas_call(
        _tc_prep_body,
        grid=(NB,),
        in_specs=[_spec_row, _spec_p0, _spec_p1],
        out_specs=_spec_row,
        out_shape=jax.ShapeDtypeStruct((N, D), jnp.float32),
    )(x, degp, degp)


def _tc_l1(agg1, hs1, degp, W1, b1, g1, c1):
    return pl.pallas_call(
        _tc_l1_body,
        grid=(NB,),
        in_specs=[
            _spec_a0, _spec_a1, _spec_row, _spec_p0, _spec_p1,
            pl.BlockSpec((D, H), _fixed),
            pl.BlockSpec((H,), _fixed1),
            pl.BlockSpec((H,), _fixed1),
            pl.BlockSpec((H,), _fixed1),
        ],
        out_specs=[_spec_row, _spec_row],
        out_shape=[jax.ShapeDtypeStruct((N, D), jnp.float32),
                   jax.ShapeDtypeStruct((N, D), jnp.float32)],
    )(agg1, agg1, hs1, degp, degp, W1, b1, g1, c1)


def _tc_l2(agg2a, agg2b, hsa, hsb, degp, W2, b2, g2, c2, W3):
    return pl.pallas_call(
        _tc_l2_body,
        grid=(NB,),
        in_specs=[
            _spec_a0, _spec_a1, _spec_a0, _spec_a1,
            _spec_row, _spec_row, _spec_p0, _spec_p1,
            pl.BlockSpec((H, H), _fixed),
            pl.BlockSpec((H,), _fixed1),
            pl.BlockSpec((H,), _fixed1),
            pl.BlockSpec((H,), _fixed1),
            pl.BlockSpec((H, D), _fixed),
        ],
        out_specs=_spec_row,
        out_shape=jax.ShapeDtypeStruct((N, D), jnp.float32),
    )(agg2a, agg2a, agg2b, agg2b, hsa, hsb, degp, degp, W2, b2, g2, c2, W3)


def _tc_fin(agg3, hs3, degp, b3):
    return pl.pallas_call(
        _tc_fin_body,
        grid=(NB,),
        in_specs=[
            _spec_a0, _spec_a1, _spec_row, _spec_p0, _spec_p1,
            pl.BlockSpec((D,), _fixed1),
        ],
        out_specs=_spec_row,
        out_shape=jax.ShapeDtypeStruct((N, D), jnp.float32),
    )(agg3, agg3, hs3, degp, degp, b3)


def kernel(x, edge_index, W1, b1, bn1_w, bn1_b, W2, b2, bn2_w, bn2_b, W3, b3):
    src = edge_index[0].astype(jnp.int32)
    dst = edge_index[1].astype(jnp.int32)
    pad = EPAD - E
    trash = N + (jnp.arange(pad, dtype=jnp.int32) % (NACC - N))
    src_p = jnp.concatenate(
        [src, jnp.zeros((pad,), jnp.int32)]).reshape(NW, CPT, CHUNK)
    dst_p = jnp.concatenate([dst, trash]).reshape(NW, CPT, CHUNK)
    src_p3 = src_p.reshape(NCHUNKS, 1, CHUNK)
    dst_p3 = dst_p.reshape(NCHUNKS, 1, CHUNK)

    degp = _sc_deg(dst_p3)
    hs1 = _tc_prep(x, degp)
    agg1 = _sc_agg(hs1, src_p3, dst_p3)
    hs2a, hs2b = _tc_l1(agg1, hs1, degp, W1, b1, bn1_w, bn1_b)
    agg2a = _sc_agg(hs2a, src_p3, dst_p3)
    agg2b = _sc_agg(hs2b, src_p3, dst_p3)
    hs3 = _tc_l2(agg2a, agg2b, hs2a, hs2b, degp, W2, b2, bn2_w, bn2_b, W3)
    agg3 = _sc_agg(hs3, src_p3, dst_p3)
    return _tc_fin(agg3, hs3, degp, b3)


# speedup vs baseline: 1.4328x; 1.4328x over previous
import functools

import jax
import jax.numpy as jnp
import numpy as np
from jax import lax
from jax.experimental import pallas as pl
from jax.experimental.pallas import tpu as pltpu
from jax.experimental.pallas import tpu_sc as plsc

N = 10000
E = 320000
D = 128
H = 256

NC = 2
NS = 16
NW = NC * NS
CHUNK = 128
CPT = 79
NCHUNKS = NW * CPT
EPAD = NCHUNKS * CHUNK
NBUF = 1
NACC = 10240
RPT = NACC // NS

BN = 1000
NB = N // BN
_BN_SCALE = np.float32(1.0 / np.sqrt(1.0 + 1e-5))

_mesh = plsc.VectorSubcoreMesh(
    core_axis_name="c", subcore_axis_name="s", num_cores=NC, num_subcores=NS)


def _zero_rows_v(rows_v, nrow, ncol):
    def body(i, _):
        for j in range(ncol // 16):
            rows_v[i, pl.ds(j * 16, 16)] = jnp.zeros((16,), jnp.float32)
        return _
    lax.fori_loop(0, nrow, body, None, unroll=False)


def _sc_deg_body(dst_hbm, out_hbm, acc, idx_v, ones_v, zero_v):
    c = lax.axis_index("c")
    s = lax.axis_index("s")
    w = c * NS + s
    one0 = jnp.where(lax.iota(jnp.int32, 16) == 0,
                     jnp.float32(1.0), jnp.float32(0.0))

    def initrow(i, _):
        ones_v[i, pl.ds(0, 16)] = one0
        zero_v[i, pl.ds(0, 16)] = jnp.zeros((16,), jnp.float32)
        return _
    lax.fori_loop(0, CHUNK, initrow, None, unroll=False)

    zb = s * RPT
    for t in range(RPT // CHUNK):
        pltpu.sync_copy(zero_v, acc.at[pl.ds(zb + t * CHUNK, CHUNK)])
    plsc.subcore_barrier()

    def chunk(k, _):
        cid = w * CPT + k
        pltpu.sync_copy(dst_hbm.at[cid], idx_v)
        pltpu.sync_copy(ones_v, acc.at[idx_v.at[0]], add=True)
        return _
    lax.fori_loop(0, CPT, chunk, None, unroll=False)
    plsc.subcore_barrier()
    pltpu.sync_copy(acc.at[pl.ds(s * RPT, RPT)],
                    out_hbm.at[c, pl.ds(s * RPT, RPT)])


@functools.partial(
    pl.kernel,
    out_type=jax.ShapeDtypeStruct((NC, NACC, 16), jnp.float32),
    mesh=_mesh,
    scratch_types=[
        pltpu.VMEM_SHARED((NACC, 16), jnp.float32),
        pltpu.VMEM((1, CHUNK), jnp.int32),
        pltpu.VMEM((CHUNK, 16), jnp.float32),
        pltpu.VMEM((CHUNK, 16), jnp.float32),
    ],
)
def _sc_deg(dst_hbm, out_hbm, acc, idx_v, ones_v, zero_v):
    _sc_deg_body(dst_hbm, out_hbm, acc, idx_v, ones_v, zero_v)


CPT0 = 80
CPT1 = 80


def _sc_agg_body(hs_hbm, src_hbm, dst_hbm, out_hbm, acc,
                 src_v, dst_v, rows_v, sem):
    c = lax.axis_index("c")
    s = lax.axis_index("s")
    w = c * NS + s
    _zero_rows_v(rows_v, CHUNK, D)
    zb = s * RPT
    for t in range(RPT // CHUNK):
        pltpu.sync_copy(rows_v, acc.at[pl.ds(zb + t * CHUNK, CHUNK)])
    plsc.subcore_barrier()

    def chunk(k, _):
        cid = w * CPT + k
        pltpu.sync_copy(src_hbm.at[cid], src_v)
        pltpu.sync_copy(dst_hbm.at[cid], dst_v)
        pltpu.async_copy(hs_hbm.at[src_v.at[0]], rows_v, sem).wait()
        pltpu.sync_copy(rows_v, acc.at[dst_v.at[0]], add=True)
        return _
    lax.fori_loop(0, CPT, chunk, None, unroll=False)
    plsc.subcore_barrier()
    pltpu.sync_copy(acc.at[pl.ds(s * RPT, RPT)],
                    out_hbm.at[c, pl.ds(s * RPT, RPT)])


@functools.partial(
    pl.kernel,
    out_type=jax.ShapeDtypeStruct((NC, NACC, D), jnp.float32),
    mesh=_mesh,
    scratch_types=[
        pltpu.VMEM_SHARED((NACC, D), jnp.float32),
        pltpu.VMEM((1, CHUNK), jnp.int32),
        pltpu.VMEM((1, CHUNK), jnp.int32),
        pltpu.VMEM((CHUNK, D), jnp.float32),
        pltpu.SemaphoreType.DMA,
    ],
)
def _sc_agg(hs_hbm, src_hbm, dst_hbm, out_hbm, acc, src_v, dst_v, rows_v, sem):
    _sc_agg_body(hs_hbm, src_hbm, dst_hbm, out_hbm, acc,
                 src_v, dst_v, rows_v, sem)


def _dinv(p0, p1):
    deg = 1.0 + p0[0][:, 0:1] + p1[0][:, 0:1]
    return lax.rsqrt(deg)


def _tc_prep_body(x_ref, p0_ref, p1_ref, hs1_ref):
    hs1_ref[...] = x_ref[...] * _dinv(p0_ref[...], p1_ref[...])


def _tc_l1_body(a0, a1, hs1, p0, p1, w1, b1, g1, c1, oa, ob):
    dinv = _dinv(p0[...], p1[...])
    comb = (a0[0] + a1[0] + hs1[...]) * dinv
    z = jnp.dot(comb, w1[...], preferred_element_type=jnp.float32) + b1[...]
    h = jax.nn.relu(z * (g1[...] * _BN_SCALE) + c1[...])
    hs2 = h * dinv
    oa[...] = hs2[:, :D]
    ob[...] = hs2[:, D:]


def _tc_l2_body(aa0, aa1, ab0, ab1, hsa, hsb, p0, p1, w2, b2, g2, c2, w3, o):
    dinv = _dinv(p0[...], p1[...])
    comb = jnp.concatenate(
        [aa0[0] + aa1[0] + hsa[...], ab0[0] + ab1[0] + hsb[...]],
        axis=1) * dinv
    z = jnp.dot(comb, w2[...], preferred_element_type=jnp.float32) + b2[...]
    h = jax.nn.relu(z * (g2[...] * _BN_SCALE) + c2[...])
    m3 = jnp.dot(h, w3[...], preferred_element_type=jnp.float32)
    o[...] = m3 * dinv


def _tc_fin_body(a0, a1, hs3, p0, p1, b3, o):
    dinv = _dinv(p0[...], p1[...])
    o[...] = (a0[0] + a1[0] + hs3[...]) * dinv + b3[...]


def _part0(i):
    return (0, i, 0)


def _part1(i):
    return (1, i, 0)


def _rows(i):
    return (i, 0)


def _fixed(i):
    return (0, 0)


def _fixed1(i):
    return (0,)


_spec_row = pl.BlockSpec((BN, D), _rows)
_spec_a0 = pl.BlockSpec((1, BN, D), _part0)
_spec_a1 = pl.BlockSpec((1, BN, D), _part1)
_spec_p0 = pl.BlockSpec((1, BN, 16), _part0)
_spec_p1 = pl.BlockSpec((1, BN, 16), _part1)


def _tc_prep(x, degp):
    return pl.pallas_call(
        _tc_prep_body,
        grid=(NB,),
        in_specs=[_spec_row, _spec_p0, _spec_p1],
        out_specs=_spec_row,
        out_shape=jax.ShapeDtypeStruct((N, D), jnp.float32),
    )(x, degp, degp)


def _tc_l1(agg1, hs1, degp, W1, b1, g1, c1):
    return pl.pallas_call(
        _tc_l1_body,
        grid=(NB,),
        in_specs=[
            _spec_a0, _spec_a1, _spec_row, _spec_p0, _spec_p1,
            pl.BlockSpec((D, H), _fixed),
            pl.BlockSpec((H,), _fixed1),
            pl.BlockSpec((H,), _fixed1),
            pl.BlockSpec((H,), _fixed1),
        ],
        out_specs=[_spec_row, _spec_row],
        out_shape=[jax.ShapeDtypeStruct((N, D), jnp.float32),
                   jax.ShapeDtypeStruct((N, D), jnp.float32)],
    )(agg1, agg1, hs1, degp, degp, W1, b1, g1, c1)


def _tc_l2(agg2a, agg2b, hsa, hsb, degp, W2, b2, g2, c2, W3):
    return pl.pallas_call(
        _tc_l2_body,
        grid=(NB,),
        in_specs=[
            _spec_a0, _spec_a1, _spec_a0, _spec_a1,
            _spec_row, _spec_row, _spec_p0, _spec_p1,
            pl.BlockSpec((H, H), _fixed),
            pl.BlockSpec((H,), _fixed1),
            pl.BlockSpec((H,), _fixed1),
            pl.BlockSpec((H,), _fixed1),
            pl.BlockSpec((H, D), _fixed),
        ],
        out_specs=_spec_row,
        out_shape=jax.ShapeDtypeStruct((N, D), jnp.float32),
    )(agg2a, agg2a, agg2b, agg2b, hsa, hsb, degp, degp, W2, b2, g2, c2, W3)


def _tc_fin(agg3, hs3, degp, b3):
    return pl.pallas_call(
        _tc_fin_body,
        grid=(NB,),
        in_specs=[
            _spec_a0, _spec_a1, _spec_row, _spec_p0, _spec_p1,
            pl.BlockSpec((D,), _fixed1),
        ],
        out_specs=_spec_row,
        out_shape=jax.ShapeDtypeStruct((N, D), jnp.float32),
    )(agg3, agg3, hs3, degp, degp, b3)


def kernel(x, edge_index, W1, b1, bn1_w, bn1_b, W2, b2, bn2_w, bn2_b, W3, b3):
    src = edge_index[0].astype(jnp.int32)
    dst = edge_index[1].astype(jnp.int32)
    pad = EPAD - E
    src_p = jnp.concatenate(
        [src, jnp.zeros((pad,), jnp.int32)]).reshape(NW, CPT, CHUNK)
    dst_p = jnp.concatenate(
        [dst, jnp.full((pad,), N, jnp.int32)]).reshape(NW, CPT, CHUNK)
    src_p3 = src_p.reshape(NCHUNKS, 1, CHUNK)
    dst_p3 = dst_p.reshape(NCHUNKS, 1, CHUNK)

    degp = _sc_deg(dst_p3)
    hs1 = _tc_prep(x, degp)
    agg1 = _sc_agg(hs1, src_p3, dst_p3)
    hs2a, hs2b = _tc_l1(agg1, hs1, degp, W1, b1, bn1_w, bn1_b)
    agg2a = _sc_agg(hs2a, src_p3, dst_p3)
    agg2b = _sc_agg(hs2b, src_p3, dst_p3)
    hs3 = _tc_l2(agg2a, agg2b, hs2a, hs2b, degp, W2, b2, bn2_w, bn2_b, W3)
    agg3 = _sc_agg(hs3, src_p3, dst_p3)
    return _tc_fin(agg3, hs3, degp, b3)


# speedup vs baseline: 2.1333x; 1.4889x over previous
import functools

import jax
import jax.numpy as jnp
import numpy as np
from jax import lax
from jax.experimental import pallas as pl
from jax.experimental.pallas import tpu as pltpu
from jax.experimental.pallas import tpu_sc as plsc

N = 10000
E = 320000
D = 128
H = 256

NC = 2
NS = 16
NW = NC * NS
CHUNK = 128
CPT = 79
NCHUNKS = NW * CPT
EPAD = NCHUNKS * CHUNK
NBUF = 1
NACC = 10240
RPT = NACC // NS

BN = 1000
NB = N // BN
_BN_SCALE = np.float32(1.0 / np.sqrt(1.0 + 1e-5))

_mesh = plsc.VectorSubcoreMesh(
    core_axis_name="c", subcore_axis_name="s", num_cores=NC, num_subcores=NS)


def _zero_rows_v(rows_v, nrow, ncol):
    def body(i, _):
        for j in range(ncol // 16):
            rows_v[i, pl.ds(j * 16, 16)] = jnp.zeros((16,), jnp.float32)
        return _
    lax.fori_loop(0, nrow, body, None, unroll=False)


def _sc_deg_body(dst_hbm, out_hbm, acc, idx_v, ones_v, zero_v):
    c = lax.axis_index("c")
    s = lax.axis_index("s")
    w = c * NS + s
    one0 = jnp.where(lax.iota(jnp.int32, 16) == 0,
                     jnp.float32(1.0), jnp.float32(0.0))

    def initrow(i, _):
        ones_v[i, pl.ds(0, 16)] = one0
        zero_v[i, pl.ds(0, 16)] = jnp.zeros((16,), jnp.float32)
        return _
    lax.fori_loop(0, CHUNK, initrow, None, unroll=False)

    zb = s * RPT
    for t in range(RPT // CHUNK):
        pltpu.sync_copy(zero_v, acc.at[pl.ds(zb + t * CHUNK, CHUNK)])
    plsc.subcore_barrier()

    def chunk(k, _):
        cid = w * CPT + k
        pltpu.sync_copy(dst_hbm.at[cid], idx_v)
        pltpu.sync_copy(ones_v, acc.at[idx_v.at[0]], add=True)
        return _
    lax.fori_loop(0, CPT, chunk, None, unroll=False)
    plsc.subcore_barrier()
    pltpu.sync_copy(acc.at[pl.ds(s * RPT, RPT)],
                    out_hbm.at[c, pl.ds(s * RPT, RPT)])


@functools.partial(
    pl.kernel,
    out_type=jax.ShapeDtypeStruct((NC, NACC, 16), jnp.float32),
    mesh=_mesh,
    scratch_types=[
        pltpu.VMEM_SHARED((NACC, 16), jnp.float32),
        pltpu.VMEM((1, CHUNK), jnp.int32),
        pltpu.VMEM((CHUNK, 16), jnp.float32),
        pltpu.VMEM((CHUNK, 16), jnp.float32),
    ],
)
def _sc_deg(dst_hbm, out_hbm, acc, idx_v, ones_v, zero_v):
    _sc_deg_body(dst_hbm, out_hbm, acc, idx_v, ones_v, zero_v)


CPT0 = 80
CPT1 = 80


def _sc_agg_body(hs_hbm, src_hbm, dst_hbm, out_hbm, acc,
                 src_v, dst_v, rows_v, sem):
    c = lax.axis_index("c")
    s = lax.axis_index("s")
    w = c * NS + s
    _zero_rows_v(rows_v, CHUNK, D)
    zb = s * RPT
    for t in range(RPT // CHUNK):
        pltpu.sync_copy(rows_v, acc.at[pl.ds(zb + t * CHUNK, CHUNK)])
    plsc.subcore_barrier()

    def chunk(k, _):
        cid = w * CPT + k
        pltpu.sync_copy(src_hbm.at[cid], src_v)
        pltpu.sync_copy(dst_hbm.at[cid], dst_v)
        pltpu.async_copy(hs_hbm.at[src_v.at[0]], rows_v, sem).wait()
        pltpu.sync_copy(rows_v, acc.at[dst_v.at[0]], add=True)
        return _
    lax.fori_loop(0, CPT, chunk, None, unroll=False)
    plsc.subcore_barrier()
    pltpu.sync_copy(acc.at[pl.ds(s * RPT, RPT)],
                    out_hbm.at[c, pl.ds(s * RPT, RPT)])


@functools.partial(
    pl.kernel,
    out_type=jax.ShapeDtypeStruct((NC, NACC, D), jnp.float32),
    mesh=_mesh,
    scratch_types=[
        pltpu.VMEM_SHARED((NACC, D), jnp.float32),
        pltpu.VMEM((1, CHUNK), jnp.int32),
        pltpu.VMEM((1, CHUNK), jnp.int32),
        pltpu.VMEM((CHUNK, D), jnp.float32),
        pltpu.SemaphoreType.DMA,
    ],
)
def _sc_agg(hs_hbm, src_hbm, dst_hbm, out_hbm, acc, src_v, dst_v, rows_v,
            sem):
    _sc_agg_body(hs_hbm, src_hbm, dst_hbm, out_hbm, acc,
                 src_v, dst_v, rows_v, sem)


def _dinv(p0, p1):
    deg = 1.0 + p0[0][:, 0:1] + p1[0][:, 0:1]
    return lax.rsqrt(deg)


def _tc_prep_body(x_ref, p0_ref, p1_ref, hs1_ref):
    hs1_ref[...] = x_ref[...] * _dinv(p0_ref[...], p1_ref[...])


def _tc_l1_body(a0, a1, hs1, p0, p1, w1, b1, g1, c1, oa, ob):
    dinv = _dinv(p0[...], p1[...])
    comb = (a0[0] + a1[0] + hs1[...]) * dinv
    z = jnp.dot(comb, w1[...], preferred_element_type=jnp.float32) + b1[...]
    h = jax.nn.relu(z * (g1[...] * _BN_SCALE) + c1[...])
    hs2 = h * dinv
    oa[...] = hs2[:, :D]
    ob[...] = hs2[:, D:]


def _tc_l2_body(aa0, aa1, ab0, ab1, hsa, hsb, p0, p1, w2, b2, g2, c2, w3, o):
    dinv = _dinv(p0[...], p1[...])
    comb = jnp.concatenate(
        [aa0[0] + aa1[0] + hsa[...], ab0[0] + ab1[0] + hsb[...]],
        axis=1) * dinv
    z = jnp.dot(comb, w2[...], preferred_element_type=jnp.float32) + b2[...]
    h = jax.nn.relu(z * (g2[...] * _BN_SCALE) + c2[...])
    m3 = jnp.dot(h, w3[...], preferred_element_type=jnp.float32)
    o[...] = m3 * dinv


def _tc_fin_body(a0, a1, hs3, p0, p1, b3, o):
    dinv = _dinv(p0[...], p1[...])
    o[...] = (a0[0] + a1[0] + hs3[...]) * dinv + b3[...]


def _part0(i):
    return (0, i, 0)


def _part1(i):
    return (1, i, 0)


def _rows(i):
    return (i, 0)


def _fixed(i):
    return (0, 0)


def _fixed1(i):
    return (0,)


_spec_row = pl.BlockSpec((BN, D), _rows)
_spec_a0 = pl.BlockSpec((1, BN, D), _part0)
_spec_a1 = pl.BlockSpec((1, BN, D), _part1)
_spec_p0 = pl.BlockSpec((1, BN, 16), _part0)
_spec_p1 = pl.BlockSpec((1, BN, 16), _part1)


def _tc_prep(x, degp):
    return pl.pallas_call(
        _tc_prep_body,
        grid=(NB,),
        in_specs=[_spec_row, _spec_p0, _spec_p1],
        out_specs=_spec_row,
        out_shape=jax.ShapeDtypeStruct((N, D), jnp.float32),
    )(x, degp, degp)


def _tc_l1(agg1, hs1, degp, W1, b1, g1, c1):
    return pl.pallas_call(
        _tc_l1_body,
        grid=(NB,),
        in_specs=[
            _spec_a0, _spec_a1, _spec_row, _spec_p0, _spec_p1,
            pl.BlockSpec((D, H), _fixed),
            pl.BlockSpec((H,), _fixed1),
            pl.BlockSpec((H,), _fixed1),
            pl.BlockSpec((H,), _fixed1),
        ],
        out_specs=[_spec_row, _spec_row],
        out_shape=[jax.ShapeDtypeStruct((N, D), jnp.float32),
                   jax.ShapeDtypeStruct((N, D), jnp.float32)],
    )(agg1, agg1, hs1, degp, degp, W1, b1, g1, c1)


def _tc_l2(agg2a, agg2b, hsa, hsb, degp, W2, b2, g2, c2, W3):
    return pl.pallas_call(
        _tc_l2_body,
        grid=(NB,),
        in_specs=[
            _spec_a0, _spec_a1, _spec_a0, _spec_a1,
            _spec_row, _spec_row, _spec_p0, _spec_p1,
            pl.BlockSpec((H, H), _fixed),
            pl.BlockSpec((H,), _fixed1),
            pl.BlockSpec((H,), _fixed1),
            pl.BlockSpec((H,), _fixed1),
            pl.BlockSpec((H, D), _fixed),
        ],
        out_specs=_spec_row,
        out_shape=jax.ShapeDtypeStruct((N, D), jnp.float32),
    )(agg2a, agg2a, agg2b, agg2b, hsa, hsb, degp, degp, W2, b2, g2, c2, W3)


def _tc_fin(agg3, hs3, degp, b3):
    return pl.pallas_call(
        _tc_fin_body,
        grid=(NB,),
        in_specs=[
            _spec_a0, _spec_a1, _spec_row, _spec_p0, _spec_p1,
            pl.BlockSpec((D,), _fixed1),
        ],
        out_specs=_spec_row,
        out_shape=jax.ShapeDtypeStruct((N, D), jnp.float32),
    )(agg3, agg3, hs3, degp, degp, b3)


def kernel(x, edge_index, W1, b1, bn1_w, bn1_b, W2, b2, bn2_w, bn2_b, W3, b3):
    src = edge_index[0].astype(jnp.int32)
    dst = edge_index[1].astype(jnp.int32)
    pad = EPAD - E
    pad_src = jnp.arange(pad, dtype=jnp.int32) % N
    pad_dst = N + (jnp.arange(pad, dtype=jnp.int32) % (NACC - N))
    src_p = jnp.concatenate([src, pad_src]).reshape(NW, CPT, CHUNK)
    dst_p = jnp.concatenate([dst, pad_dst]).reshape(NW, CPT, CHUNK)
    src_p3 = src_p.reshape(NCHUNKS, 1, CHUNK)
    dst_p3 = dst_p.reshape(NCHUNKS, 1, CHUNK)

    degp = _sc_deg(dst_p3)
    hs1 = _tc_prep(x, degp)
    agg1 = _sc_agg(hs1, src_p3, dst_p3)
    hs2a, hs2b = _tc_l1(agg1, hs1, degp, W1, b1, bn1_w, bn1_b)
    agg2a = _sc_agg(hs2a, src_p3, dst_p3)
    agg2b = _sc_agg(hs2b, src_p3, dst_p3)
    hs3 = _tc_l2(agg2a, agg2b, hs2a, hs2b, degp, W2, b2, bn2_w, bn2_b, W3)
    agg3 = _sc_agg(hs3, src_p3, dst_p3)
    return _tc_fin(agg3, hs3, degp, b3)


# speedup vs baseline: 2.7542x; 1.2910x over previous
import functools

import jax
import jax.numpy as jnp
import numpy as np
from jax import lax
from jax.experimental import pallas as pl
from jax.experimental.pallas import tpu as pltpu
from jax.experimental.pallas import tpu_sc as plsc

N = 10000
E = 320000
D = 128
H = 256

NC = 2
NS = 16
NW = NC * NS
CHUNK = 256
CPT = 40
NCHUNKS = NW * CPT
EPAD = NCHUNKS * CHUNK
NBUF = 1
NACC = 10240
RPT = NACC // NS

BN = 1000
NB = N // BN
_BN_SCALE = np.float32(1.0 / np.sqrt(1.0 + 1e-5))

_mesh = plsc.VectorSubcoreMesh(
    core_axis_name="c", subcore_axis_name="s", num_cores=NC, num_subcores=NS)


def _zero_rows_v(rows_v, nrow, ncol):
    def body(i, _):
        for j in range(ncol // 16):
            rows_v[i, pl.ds(j * 16, 16)] = jnp.zeros((16,), jnp.float32)
        return _
    lax.fori_loop(0, nrow, body, None, unroll=False)


def _sc_deg_body(dst_hbm, out_hbm, acc, idx_v, ones_v, zero_v):
    c = lax.axis_index("c")
    s = lax.axis_index("s")
    w = c * NS + s
    one0 = jnp.where(lax.iota(jnp.int32, 16) == 0,
                     jnp.float32(1.0), jnp.float32(0.0))

    def initrow(i, _):
        ones_v[i, pl.ds(0, 16)] = one0
        zero_v[i, pl.ds(0, 16)] = jnp.zeros((16,), jnp.float32)
        return _
    lax.fori_loop(0, CHUNK, initrow, None, unroll=False)

    zb = s * RPT
    for t in range(RPT // CHUNK):
        pltpu.sync_copy(zero_v, acc.at[pl.ds(zb + t * CHUNK, CHUNK)])
    if RPT % CHUNK:
        pltpu.sync_copy(
            zero_v.at[pl.ds(0, RPT % CHUNK)],
            acc.at[pl.ds(zb + (RPT // CHUNK) * CHUNK, RPT % CHUNK)])
    plsc.subcore_barrier()

    def chunk(k, _):
        cid = w * CPT + k
        pltpu.sync_copy(dst_hbm.at[cid], idx_v)
        pltpu.sync_copy(ones_v, acc.at[idx_v.at[0]], add=True)
        return _
    lax.fori_loop(0, CPT, chunk, None, unroll=False)
    plsc.subcore_barrier()
    pltpu.sync_copy(acc.at[pl.ds(s * RPT, RPT)],
                    out_hbm.at[c, pl.ds(s * RPT, RPT)])


@functools.partial(
    pl.kernel,
    out_type=jax.ShapeDtypeStruct((NC, NACC, 16), jnp.float32),
    mesh=_mesh,
    scratch_types=[
        pltpu.VMEM_SHARED((NACC, 16), jnp.float32),
        pltpu.VMEM((1, CHUNK), jnp.int32),
        pltpu.VMEM((CHUNK, 16), jnp.float32),
        pltpu.VMEM((CHUNK, 16), jnp.float32),
    ],
)
def _sc_deg(dst_hbm, out_hbm, acc, idx_v, ones_v, zero_v):
    _sc_deg_body(dst_hbm, out_hbm, acc, idx_v, ones_v, zero_v)


CPT0 = 80
CPT1 = 80


def _sc_agg_body(hs_hbm, src_hbm, dst_hbm, out_hbm, acc,
                 src_v, dst_v, rows_v, sem):
    c = lax.axis_index("c")
    s = lax.axis_index("s")
    w = c * NS + s
    _zero_rows_v(rows_v, CHUNK, D)
    zb = s * RPT
    for t in range(RPT // CHUNK):
        pltpu.sync_copy(rows_v, acc.at[pl.ds(zb + t * CHUNK, CHUNK)])
    if RPT % CHUNK:
        pltpu.sync_copy(
            rows_v.at[pl.ds(0, RPT % CHUNK)],
            acc.at[pl.ds(zb + (RPT // CHUNK) * CHUNK, RPT % CHUNK)])
    plsc.subcore_barrier()

    def chunk(k, _):
        cid = w * CPT + k
        pltpu.sync_copy(src_hbm.at[cid], src_v)
        pltpu.sync_copy(dst_hbm.at[cid], dst_v)
        pltpu.async_copy(hs_hbm.at[src_v.at[0]], rows_v, sem).wait()
        pltpu.sync_copy(rows_v, acc.at[dst_v.at[0]], add=True)
        return _
    lax.fori_loop(0, CPT, chunk, None, unroll=False)
    plsc.subcore_barrier()
    pltpu.sync_copy(acc.at[pl.ds(s * RPT, RPT)],
                    out_hbm.at[c, pl.ds(s * RPT, RPT)])


@functools.partial(
    pl.kernel,
    out_type=jax.ShapeDtypeStruct((NC, NACC, D), jnp.float32),
    mesh=_mesh,
    scratch_types=[
        pltpu.VMEM_SHARED((NACC, D), jnp.float32),
        pltpu.VMEM((1, CHUNK), jnp.int32),
        pltpu.VMEM((1, CHUNK), jnp.int32),
        pltpu.VMEM((CHUNK, D), jnp.float32),
        pltpu.SemaphoreType.DMA,
    ],
)
def _sc_agg(hs_hbm, src_hbm, dst_hbm, out_hbm, acc, src_v, dst_v, rows_v,
            sem):
    _sc_agg_body(hs_hbm, src_hbm, dst_hbm, out_hbm, acc,
                 src_v, dst_v, rows_v, sem)


def _dinv(p0, p1):
    deg = 1.0 + p0[0][:, 0:1] + p1[0][:, 0:1]
    return lax.rsqrt(deg)


def _tc_prep_body(x_ref, p0_ref, p1_ref, hs1_ref):
    hs1_ref[...] = x_ref[...] * _dinv(p0_ref[...], p1_ref[...])


def _tc_l1_body(a0, a1, hs1, p0, p1, w1, b1, g1, c1, oa, ob):
    dinv = _dinv(p0[...], p1[...])
    comb = (a0[0] + a1[0] + hs1[...]) * dinv
    z = jnp.dot(comb, w1[...], preferred_element_type=jnp.float32) + b1[...]
    h = jax.nn.relu(z * (g1[...] * _BN_SCALE) + c1[...])
    hs2 = h * dinv
    oa[...] = hs2[:, :D]
    ob[...] = hs2[:, D:]


def _tc_l2_body(aa0, aa1, ab0, ab1, hsa, hsb, p0, p1, w2, b2, g2, c2, w3, o):
    dinv = _dinv(p0[...], p1[...])
    comb = jnp.concatenate(
        [aa0[0] + aa1[0] + hsa[...], ab0[0] + ab1[0] + hsb[...]],
        axis=1) * dinv
    z = jnp.dot(comb, w2[...], preferred_element_type=jnp.float32) + b2[...]
    h = jax.nn.relu(z * (g2[...] * _BN_SCALE) + c2[...])
    m3 = jnp.dot(h, w3[...], preferred_element_type=jnp.float32)
    o[...] = m3 * dinv


def _tc_fin_body(a0, a1, hs3, p0, p1, b3, o):
    dinv = _dinv(p0[...], p1[...])
    o[...] = (a0[0] + a1[0] + hs3[...]) * dinv + b3[...]


def _part0(i):
    return (0, i, 0)


def _part1(i):
    return (1, i, 0)


def _rows(i):
    return (i, 0)


def _fixed(i):
    return (0, 0)


def _fixed1(i):
    return (0,)


_spec_row = pl.BlockSpec((BN, D), _rows)
_spec_a0 = pl.BlockSpec((1, BN, D), _part0)
_spec_a1 = pl.BlockSpec((1, BN, D), _part1)
_spec_p0 = pl.BlockSpec((1, BN, 16), _part0)
_spec_p1 = pl.BlockSpec((1, BN, 16), _part1)


def _tc_prep(x, degp):
    return pl.pallas_call(
        _tc_prep_body,
        grid=(NB,),
        in_specs=[_spec_row, _spec_p0, _spec_p1],
        out_specs=_spec_row,
        out_shape=jax.ShapeDtypeStruct((N, D), jnp.float32),
    )(x, degp, degp)


def _tc_l1(agg1, hs1, degp, W1, b1, g1, c1):
    return pl.pallas_call(
        _tc_l1_body,
        grid=(NB,),
        in_specs=[
            _spec_a0, _spec_a1, _spec_row, _spec_p0, _spec_p1,
            pl.BlockSpec((D, H), _fixed),
            pl.BlockSpec((H,), _fixed1),
            pl.BlockSpec((H,), _fixed1),
            pl.BlockSpec((H,), _fixed1),
        ],
        out_specs=[_spec_row, _spec_row],
        out_shape=[jax.ShapeDtypeStruct((N, D), jnp.float32),
                   jax.ShapeDtypeStruct((N, D), jnp.float32)],
    )(agg1, agg1, hs1, degp, degp, W1, b1, g1, c1)


def _tc_l2(agg2a, agg2b, hsa, hsb, degp, W2, b2, g2, c2, W3):
    return pl.pallas_call(
        _tc_l2_body,
        grid=(NB,),
        in_specs=[
            _spec_a0, _spec_a1, _spec_a0, _spec_a1,
            _spec_row, _spec_row, _spec_p0, _spec_p1,
            pl.BlockSpec((H, H), _fixed),
            pl.BlockSpec((H,), _fixed1),
            pl.BlockSpec((H,), _fixed1),
            pl.BlockSpec((H,), _fixed1),
            pl.BlockSpec((H, D), _fixed),
        ],
        out_specs=_spec_row,
        out_shape=jax.ShapeDtypeStruct((N, D), jnp.float32),
    )(agg2a, agg2a, agg2b, agg2b, hsa, hsb, degp, degp, W2, b2, g2, c2, W3)


def _tc_fin(agg3, hs3, degp, b3):
    return pl.pallas_call(
        _tc_fin_body,
        grid=(NB,),
        in_specs=[
            _spec_a0, _spec_a1, _spec_row, _spec_p0, _spec_p1,
            pl.BlockSpec((D,), _fixed1),
        ],
        out_specs=_spec_row,
        out_shape=jax.ShapeDtypeStruct((N, D), jnp.float32),
    )(agg3, agg3, hs3, degp, degp, b3)


def kernel(x, edge_index, W1, b1, bn1_w, bn1_b, W2, b2, bn2_w, bn2_b, W3, b3):
    src = edge_index[0].astype(jnp.int32)
    dst = edge_index[1].astype(jnp.int32)
    pad = EPAD - E
    pad_src = jnp.arange(pad, dtype=jnp.int32) % N
    pad_dst = N + (jnp.arange(pad, dtype=jnp.int32) % (NACC - N))
    src_p = jnp.concatenate([src, pad_src]).reshape(NW, CPT, CHUNK)
    dst_p = jnp.concatenate([dst, pad_dst]).reshape(NW, CPT, CHUNK)
    src_p3 = src_p.reshape(NCHUNKS, 1, CHUNK)
    dst_p3 = dst_p.reshape(NCHUNKS, 1, CHUNK)

    degp = _sc_deg(dst_p3)
    hs1 = _tc_prep(x, degp)
    agg1 = _sc_agg(hs1, src_p3, dst_p3)
    hs2a, hs2b = _tc_l1(agg1, hs1, degp, W1, b1, bn1_w, bn1_b)
    agg2a = _sc_agg(hs2a, src_p3, dst_p3)
    agg2b = _sc_agg(hs2b, src_p3, dst_p3)
    hs3 = _tc_l2(agg2a, agg2b, hs2a, hs2b, degp, W2, b2, bn2_w, bn2_b, W3)
    agg3 = _sc_agg(hs3, src_p3, dst_p3)
    return _tc_fin(agg3, hs3, degp, b3)


# speedup vs baseline: 2.8851x; 1.0475x over previous
import functools

import jax
import jax.numpy as jnp
import numpy as np
from jax import lax
from jax.experimental import pallas as pl
from jax.experimental.pallas import tpu as pltpu
from jax.experimental.pallas import tpu_sc as plsc

N = 10000
E = 320000
D = 128
H = 256

NC = 2
NS = 16
NW = NC * NS
CHUNK = 320
CPT = 32
NCHUNKS = NW * CPT
EPAD = NCHUNKS * CHUNK
NBUF = 1
NACC = 10240
RPT = NACC // NS

BN = 1000
NB = N // BN
_BN_SCALE = np.float32(1.0 / np.sqrt(1.0 + 1e-5))

_mesh = plsc.VectorSubcoreMesh(
    core_axis_name="c", subcore_axis_name="s", num_cores=NC, num_subcores=NS)


def _zero_rows_v(rows_v, nrow, ncol):
    def body(i, _):
        for j in range(ncol // 16):
            rows_v[i, pl.ds(j * 16, 16)] = jnp.zeros((16,), jnp.float32)
        return _
    lax.fori_loop(0, nrow, body, None, unroll=False)


def _sc_deg_body(dst_hbm, out_hbm, acc, idx_v, ones_v, zero_v):
    c = lax.axis_index("c")
    s = lax.axis_index("s")
    w = c * NS + s
    one0 = jnp.where(lax.iota(jnp.int32, 16) == 0,
                     jnp.float32(1.0), jnp.float32(0.0))

    def initrow(i, _):
        ones_v[i, pl.ds(0, 16)] = one0
        zero_v[i, pl.ds(0, 16)] = jnp.zeros((16,), jnp.float32)
        return _
    lax.fori_loop(0, CHUNK, initrow, None, unroll=False)

    zb = s * RPT
    for t in range(RPT // CHUNK):
        pltpu.sync_copy(zero_v, acc.at[pl.ds(zb + t * CHUNK, CHUNK)])
    if RPT % CHUNK:
        pltpu.sync_copy(
            zero_v.at[pl.ds(0, RPT % CHUNK)],
            acc.at[pl.ds(zb + (RPT // CHUNK) * CHUNK, RPT % CHUNK)])
    plsc.subcore_barrier()

    def chunk(k, _):
        cid = w * CPT + k
        pltpu.sync_copy(dst_hbm.at[cid], idx_v)
        pltpu.sync_copy(ones_v, acc.at[idx_v.at[0]], add=True)
        return _
    lax.fori_loop(0, CPT, chunk, None, unroll=False)
    plsc.subcore_barrier()
    pltpu.sync_copy(acc.at[pl.ds(s * RPT, RPT)],
                    out_hbm.at[c, pl.ds(s * RPT, RPT)])


@functools.partial(
    pl.kernel,
    out_type=jax.ShapeDtypeStruct((NC, NACC, 16), jnp.float32),
    mesh=_mesh,
    scratch_types=[
        pltpu.VMEM_SHARED((NACC, 16), jnp.float32),
        pltpu.VMEM((1, CHUNK), jnp.int32),
        pltpu.VMEM((CHUNK, 16), jnp.float32),
        pltpu.VMEM((CHUNK, 16), jnp.float32),
    ],
)
def _sc_deg(dst_hbm, out_hbm, acc, idx_v, ones_v, zero_v):
    _sc_deg_body(dst_hbm, out_hbm, acc, idx_v, ones_v, zero_v)


CPT0 = 80
CPT1 = 80


def _sc_agg_body(hs_hbm, src_hbm, dst_hbm, out_hbm, acc,
                 src_v, dst_v, rows_v, sem):
    c = lax.axis_index("c")
    s = lax.axis_index("s")
    w = c * NS + s
    _zero_rows_v(rows_v, CHUNK, D)
    zb = s * RPT
    for t in range(RPT // CHUNK):
        pltpu.sync_copy(rows_v, acc.at[pl.ds(zb + t * CHUNK, CHUNK)])
    if RPT % CHUNK:
        pltpu.sync_copy(
            rows_v.at[pl.ds(0, RPT % CHUNK)],
            acc.at[pl.ds(zb + (RPT // CHUNK) * CHUNK, RPT % CHUNK)])
    plsc.subcore_barrier()

    def chunk(k, _):
        cid = w * CPT + k
        pltpu.sync_copy(src_hbm.at[cid], src_v)
        pltpu.sync_copy(dst_hbm.at[cid], dst_v)
        pltpu.async_copy(hs_hbm.at[src_v.at[0]], rows_v, sem).wait()
        pltpu.sync_copy(rows_v, acc.at[dst_v.at[0]], add=True)
        return _
    lax.fori_loop(0, CPT, chunk, None, unroll=False)
    plsc.subcore_barrier()
    pltpu.sync_copy(acc.at[pl.ds(s * RPT, RPT)],
                    out_hbm.at[c, pl.ds(s * RPT, RPT)])


@functools.partial(
    pl.kernel,
    out_type=jax.ShapeDtypeStruct((NC, NACC, D), jnp.float32),
    mesh=_mesh,
    scratch_types=[
        pltpu.VMEM_SHARED((NACC, D), jnp.float32),
        pltpu.VMEM((1, CHUNK), jnp.int32),
        pltpu.VMEM((1, CHUNK), jnp.int32),
        pltpu.VMEM((CHUNK, D), jnp.float32),
        pltpu.SemaphoreType.DMA,
    ],
)
def _sc_agg(hs_hbm, src_hbm, dst_hbm, out_hbm, acc, src_v, dst_v, rows_v,
            sem):
    _sc_agg_body(hs_hbm, src_hbm, dst_hbm, out_hbm, acc,
                 src_v, dst_v, rows_v, sem)


def _dinv(p0, p1):
    deg = 1.0 + p0[0][:, 0:1] + p1[0][:, 0:1]
    return lax.rsqrt(deg)


def _tc_prep_body(x_ref, p0_ref, p1_ref, hs1_ref):
    hs1_ref[...] = x_ref[...] * _dinv(p0_ref[...], p1_ref[...])


def _tc_l1_body(a0, a1, hs1, p0, p1, w1, b1, g1, c1, oa, ob):
    dinv = _dinv(p0[...], p1[...])
    comb = (a0[0] + a1[0] + hs1[...]) * dinv
    z = jnp.dot(comb, w1[...], preferred_element_type=jnp.float32) + b1[...]
    h = jax.nn.relu(z * (g1[...] * _BN_SCALE) + c1[...])
    hs2 = h * dinv
    oa[...] = hs2[:, :D]
    ob[...] = hs2[:, D:]


def _tc_l2_body(aa0, aa1, ab0, ab1, hsa, hsb, p0, p1, w2, b2, g2, c2, w3, o):
    dinv = _dinv(p0[...], p1[...])
    comb = jnp.concatenate(
        [aa0[0] + aa1[0] + hsa[...], ab0[0] + ab1[0] + hsb[...]],
        axis=1) * dinv
    z = jnp.dot(comb, w2[...], preferred_element_type=jnp.float32) + b2[...]
    h = jax.nn.relu(z * (g2[...] * _BN_SCALE) + c2[...])
    m3 = jnp.dot(h, w3[...], preferred_element_type=jnp.float32)
    o[...] = m3 * dinv


def _tc_fin_body(a0, a1, hs3, p0, p1, b3, o):
    dinv = _dinv(p0[...], p1[...])
    o[...] = (a0[0] + a1[0] + hs3[...]) * dinv + b3[...]


def _part0(i):
    return (0, i, 0)


def _part1(i):
    return (1, i, 0)


def _rows(i):
    return (i, 0)


def _fixed(i):
    return (0, 0)


def _fixed1(i):
    return (0,)


_spec_row = pl.BlockSpec((BN, D), _rows)
_spec_a0 = pl.BlockSpec((1, BN, D), _part0)
_spec_a1 = pl.BlockSpec((1, BN, D), _part1)
_spec_p0 = pl.BlockSpec((1, BN, 16), _part0)
_spec_p1 = pl.BlockSpec((1, BN, 16), _part1)


def _tc_prep(x, degp):
    return pl.pallas_call(
        _tc_prep_body,
        grid=(NB,),
        in_specs=[_spec_row, _spec_p0, _spec_p1],
        out_specs=_spec_row,
        out_shape=jax.ShapeDtypeStruct((N, D), jnp.float32),
    )(x, degp, degp)


def _tc_l1(agg1, hs1, degp, W1, b1, g1, c1):
    return pl.pallas_call(
        _tc_l1_body,
        grid=(NB,),
        in_specs=[
            _spec_a0, _spec_a1, _spec_row, _spec_p0, _spec_p1,
            pl.BlockSpec((D, H), _fixed),
            pl.BlockSpec((H,), _fixed1),
            pl.BlockSpec((H,), _fixed1),
            pl.BlockSpec((H,), _fixed1),
        ],
        out_specs=[_spec_row, _spec_row],
        out_shape=[jax.ShapeDtypeStruct((N, D), jnp.float32),
                   jax.ShapeDtypeStruct((N, D), jnp.float32)],
    )(agg1, agg1, hs1, degp, degp, W1, b1, g1, c1)


def _tc_l2(agg2a, agg2b, hsa, hsb, degp, W2, b2, g2, c2, W3):
    return pl.pallas_call(
        _tc_l2_body,
        grid=(NB,),
        in_specs=[
            _spec_a0, _spec_a1, _spec_a0, _spec_a1,
            _spec_row, _spec_row, _spec_p0, _spec_p1,
            pl.BlockSpec((H, H), _fixed),
            pl.BlockSpec((H,), _fixed1),
            pl.BlockSpec((H,), _fixed1),
            pl.BlockSpec((H,), _fixed1),
            pl.BlockSpec((H, D), _fixed),
        ],
        out_specs=_spec_row,
        out_shape=jax.ShapeDtypeStruct((N, D), jnp.float32),
    )(agg2a, agg2a, agg2b, agg2b, hsa, hsb, degp, degp, W2, b2, g2, c2, W3)


def _tc_fin(agg3, hs3, degp, b3):
    return pl.pallas_call(
        _tc_fin_body,
        grid=(NB,),
        in_specs=[
            _spec_a0, _spec_a1, _spec_row, _spec_p0, _spec_p1,
            pl.BlockSpec((D,), _fixed1),
        ],
        out_specs=_spec_row,
        out_shape=jax.ShapeDtypeStruct((N, D), jnp.float32),
    )(agg3, agg3, hs3, degp, degp, b3)


def kernel(x, edge_index, W1, b1, bn1_w, bn1_b, W2, b2, bn2_w, bn2_b, W3, b3):
    src = edge_index[0].astype(jnp.int32)
    dst = edge_index[1].astype(jnp.int32)
    pad = EPAD - E
    pad_src = jnp.arange(pad, dtype=jnp.int32) % N
    pad_dst = N + (jnp.arange(pad, dtype=jnp.int32) % (NACC - N))
    src_p = jnp.concatenate([src, pad_src]).reshape(NW, CPT, CHUNK)
    dst_p = jnp.concatenate([dst, pad_dst]).reshape(NW, CPT, CHUNK)
    src_p3 = src_p.reshape(NCHUNKS, 1, CHUNK)
    dst_p3 = dst_p.reshape(NCHUNKS, 1, CHUNK)

    degp = _sc_deg(dst_p3)
    hs1 = _tc_prep(x, degp)
    agg1 = _sc_agg(hs1, src_p3, dst_p3)
    hs2a, hs2b = _tc_l1(agg1, hs1, degp, W1, b1, bn1_w, bn1_b)
    agg2a = _sc_agg(hs2a, src_p3, dst_p3)
    agg2b = _sc_agg(hs2b, src_p3, dst_p3)
    hs3 = _tc_l2(agg2a, agg2b, hs2a, hs2b, degp, W2, b2, bn2_w, bn2_b, W3)
    agg3 = _sc_agg(hs3, src_p3, dst_p3)
    return _tc_fin(agg3, hs3, degp, b3)
